# Initial kernel scaffold; baseline (speedup 1.0000x reference)
import jax
import jax.numpy as jnp
from jax.experimental import pallas as pl


def kernel(hyperedge, hyper_node, ve_affiliation, W_v2e, b_v2e, W_upd, b_upd):
    raise NotImplementedError("write your pallas kernel here")


# speedup vs baseline: 3.2530x; 3.2530x over previous
import functools

import jax
import jax.numpy as jnp
from jax import lax
from jax.experimental import pallas as pl
from jax.experimental.pallas import tpu as pltpu
from jax.experimental.pallas import tpu_sc as plsc

N = 320000
M = 10000
D = 128
MP = 10240

NC, NS = 2, 16
PER_TILE = N // (NC * NS)
CHUNK = 200
SUB = 40
NSUB = CHUNK // SUB
NCHUNK = PER_TILE // CHUNK
OUT_ROWS = MP // NS
CROWS = 8


def _mm_relu_body(x_ref, w_ref, b_ref, o_ref):
    o_ref[...] = jnp.maximum(
        jnp.dot(x_ref[...], w_ref[...], preferred_element_type=jnp.float32)
        + b_ref[...], 0.0)


def _node_transform(hyper_node, W_v2e, b_v2e):
    BN = 3200
    grid = (N // BN,)
    return pl.pallas_call(
        _mm_relu_body,
        grid=grid,
        in_specs=[
            pl.BlockSpec((BN, D), lambda i: (i, 0)),
            pl.BlockSpec((D, D), lambda i: (0, 0)),
            pl.BlockSpec((1, D), lambda i: (0, 0)),
        ],
        out_specs=pl.BlockSpec((BN, D), lambda i: (i, 0)),
        out_shape=jax.ShapeDtypeStruct((N, D), jnp.float32),
    )(hyper_node, W_v2e, b_v2e.reshape(1, D))


def _scatter_body(ni_hbm, idx_hbm, zsum_hbm,
                  sum_out, cnt_out,
                  rows_v, idx_v0, idx_v1, idx_v2, idx_v3, idx_v4,
                  ones_v, zc_v, sum_acc, cnt_acc):
    c = lax.axis_index("c")
    s = lax.axis_index("s")
    idx_refs = (idx_v0, idx_v1, idx_v2, idx_v3, idx_v4)

    one16 = jnp.ones((16,), jnp.float32)
    zero16 = jnp.zeros((16,), jnp.float32)
    ones_v[pl.ds(0, 16)] = one16
    ones_v[pl.ds(16, 16)] = one16
    ones_v[pl.ds(SUB - 16, 16)] = one16
    for k in range(OUT_ROWS // 16):
        zc_v[pl.ds(16 * k, 16)] = zero16

    r0 = pl.multiple_of(s * OUT_ROWS, 8)
    pltpu.sync_copy(zsum_hbm.at[pl.ds(r0, OUT_ROWS)],
                    sum_acc.at[pl.ds(r0, OUT_ROWS)])
    pltpu.sync_copy(zc_v, cnt_acc.at[pl.ds(r0, OUT_ROWS)])
    plsc.subcore_barrier()

    node_base = c * (N // NC) + s * PER_TILE

    @pl.loop(0, NCHUNK)
    def body(j):
        base = pl.multiple_of(node_base + j * CHUNK, 8)
        pltpu.sync_copy(ni_hbm.at[pl.ds(base, CHUNK)], rows_v)
        for t in range(NSUB):
            pltpu.sync_copy(idx_hbm.at[pl.ds(base + t * SUB, SUB)],
                            idx_refs[t])
        for t in range(NSUB):
            pltpu.sync_copy(rows_v.at[pl.ds(t * SUB, SUB)],
                            sum_acc.at[idx_refs[t]], add=True)
            pltpu.sync_copy(ones_v, cnt_acc.at[idx_refs[t]], add=True)

    plsc.subcore_barrier()

    pltpu.sync_copy(sum_acc.at[pl.ds(r0, OUT_ROWS)],
                    sum_out.at[c, pl.ds(r0, OUT_ROWS)])
    pltpu.sync_copy(cnt_acc.at[pl.ds(r0, OUT_ROWS)], zc_v)
    for k in range(OUT_ROWS // 16):
        rows_v[k // 8, pl.ds((k % 8) * 16, 16)] = zc_v[pl.ds(16 * k, 16)]
    cr0 = pl.multiple_of(s * CROWS, 8)
    pltpu.sync_copy(rows_v.at[pl.ds(0, CROWS)],
                    cnt_out.at[c, pl.ds(cr0, CROWS)])


def _segment_sums(node_info, idx, zsum):
    mesh = plsc.VectorSubcoreMesh(core_axis_name="c", subcore_axis_name="s")
    f = functools.partial(
        pl.kernel,
        out_type=(jax.ShapeDtypeStruct((NC, MP, D), jnp.float32),
                  jax.ShapeDtypeStruct((NC, NS * CROWS, D), jnp.float32)),
        mesh=mesh,
        scratch_types=[
            pltpu.VMEM((CHUNK, D), jnp.float32),
        ] + [pltpu.VMEM((SUB,), jnp.int32) for _ in range(NSUB)] + [
            pltpu.VMEM((SUB,), jnp.float32),
            pltpu.VMEM((OUT_ROWS,), jnp.float32),
            pltpu.VMEM_SHARED((MP, D), jnp.float32),
            pltpu.VMEM_SHARED((MP,), jnp.float32),
        ],
    )(_scatter_body)
    return f(node_info, idx, zsum)


def _update_body(sums_ref, cnt_ref, he_ref, wa_ref, wb_ref, b_ref, o_ref):
    ssum = sums_ref[0] + sums_ref[1]
    cnt = jnp.sum(cnt_ref[...], axis=1, keepdims=True)
    mean = ssum / jnp.maximum(cnt, 1.0)
    h = jnp.dot(mean, wa_ref[...], preferred_element_type=jnp.float32)
    h = h + jnp.dot(he_ref[...], wb_ref[...], preferred_element_type=jnp.float32)
    h = jnp.maximum(h + b_ref[...], 0.0)
    nrm = jnp.sqrt(jnp.sum(h * h, axis=-1, keepdims=True))
    o_ref[...] = h / jnp.maximum(nrm, 1e-12)


def _update(sums, cntsT, hyperedge, W_upd, b_upd):
    BR = 2000
    grid = (M // BR,)
    return pl.pallas_call(
        _update_body,
        grid=grid,
        in_specs=[
            pl.BlockSpec((NC, BR, D), lambda i: (0, i, 0)),
            pl.BlockSpec((BR, NC), lambda i: (i, 0)),
            pl.BlockSpec((BR, D), lambda i: (i, 0)),
            pl.BlockSpec((D, D), lambda i: (0, 0)),
            pl.BlockSpec((D, D), lambda i: (0, 0)),
            pl.BlockSpec((1, D), lambda i: (0, 0)),
        ],
        out_specs=pl.BlockSpec((BR, D), lambda i: (i, 0)),
        out_shape=jax.ShapeDtypeStruct((M, D), jnp.float32),
    )(sums, cntsT, hyperedge, W_upd[:D], W_upd[D:], b_upd.reshape(1, D))


def kernel(hyperedge, hyper_node, ve_affiliation, W_v2e, b_v2e, W_upd, b_upd):
    assert hyper_node.shape == (N, D) and hyperedge.shape == (M, D)
    node_info = _node_transform(hyper_node, W_v2e, b_v2e)
    idx = ve_affiliation[0]
    zsum = jnp.zeros((MP, D), jnp.float32)
    sums, cnts_wide = _segment_sums(node_info, idx, zsum)
    cntsT = (cnts_wide.reshape(NC, NS, CROWS * D)[:, :, :OUT_ROWS]
             .reshape(NC, MP).T)
    return _update(sums, cntsT, hyperedge, W_upd, b_upd)


# speedup vs baseline: 5.1796x; 1.5923x over previous
import functools

import jax
import jax.numpy as jnp
from jax import lax
from jax.experimental import pallas as pl
from jax.experimental.pallas import tpu as pltpu
from jax.experimental.pallas import tpu_sc as plsc

N = 320000
M = 10000
D = 128
MP = 10240

NC, NS = 2, 16
PER_TILE = N // (NC * NS)
CHUNK = 128
NFULL = PER_TILE // CHUNK
TAIL = PER_TILE - NFULL * CHUNK
OUT_ROWS = MP // NS
CROWS = 8


def _mm_relu_body(x_ref, w_ref, b_ref, o_ref):
    o_ref[...] = jnp.maximum(
        jnp.dot(x_ref[...], w_ref[...], preferred_element_type=jnp.float32)
        + b_ref[...], 0.0)


def _node_transform(hyper_node, W_v2e, b_v2e):
    BN = 3200
    grid = (N // BN,)
    return pl.pallas_call(
        _mm_relu_body,
        grid=grid,
        in_specs=[
            pl.BlockSpec((BN, D), lambda i: (i, 0)),
            pl.BlockSpec((D, D), lambda i: (0, 0)),
            pl.BlockSpec((1, D), lambda i: (0, 0)),
        ],
        out_specs=pl.BlockSpec((BN, D), lambda i: (i, 0)),
        out_shape=jax.ShapeDtypeStruct((N, D), jnp.float32),
    )(hyper_node, W_v2e, b_v2e.reshape(1, D))


def _scatter_body(ni_hbm, idx_hbm, zsum_hbm,
                  sum_out, cnt_out,
                  rows_v0, rows_v1, idx_v0, idx_v1, idxt_v,
                  ones_v, zc_v, sum_acc, cnt_acc,
                  rsem0, rsem1, isem0, isem1, ssem0, ssem1, csem0, csem1):
    c = lax.axis_index("c")
    s = lax.axis_index("s")
    rows = (rows_v0, rows_v1)
    idxs = (idx_v0, idx_v1)
    rsem = (rsem0, rsem1)
    isem = (isem0, isem1)
    ssem = (ssem0, ssem1)
    csem = (csem0, csem1)

    one16 = jnp.ones((16,), jnp.float32)
    zero16 = jnp.zeros((16,), jnp.float32)
    for k in range(CHUNK // 16):
        ones_v[pl.ds(16 * k, 16)] = one16
    for k in range(OUT_ROWS // 16):
        zc_v[pl.ds(16 * k, 16)] = zero16

    r0 = pl.multiple_of(s * OUT_ROWS, 8)
    pltpu.sync_copy(zsum_hbm.at[pl.ds(r0, OUT_ROWS)],
                    sum_acc.at[pl.ds(r0, OUT_ROWS)])
    pltpu.sync_copy(zc_v, cnt_acc.at[pl.ds(r0, OUT_ROWS)])
    plsc.subcore_barrier()

    node_base = c * (N // NC) + s * PER_TILE

    def start_fetch(k, b):
        kk = jnp.where(k < NFULL, k, 0)
        base = pl.multiple_of(node_base + kk * CHUNK, 8)
        pltpu.make_async_copy(ni_hbm.at[pl.ds(base, CHUNK)],
                              rows[b], rsem[b]).start()
        pltpu.make_async_copy(idx_hbm.at[pl.ds(base, CHUNK)],
                              idxs[b], isem[b]).start()

    def wait_fetch(b):
        pltpu.make_async_copy(ni_hbm.at[pl.ds(0, CHUNK)],
                              rows[b], rsem[b]).wait()
        pltpu.make_async_copy(idx_hbm.at[pl.ds(0, CHUNK)],
                              idxs[b], isem[b]).wait()

    def step(k, b):
        wait_fetch(b)
        sum_d = pltpu.make_async_copy(rows[b], sum_acc.at[idxs[b]], ssem[b])
        cnt_d = pltpu.make_async_copy(ones_v, cnt_acc.at[idxs[b]], csem[b])
        sum_d.start(add=True)
        cnt_d.start(add=True)
        sum_d.wait()
        cnt_d.wait()
        start_fetch(k + 2, b)

    start_fetch(0, 0)
    start_fetch(1, 1)
    step(0, 0)
    step(1, 1)

    @pl.loop(0, (NFULL - 2) // 2)
    def body(g):
        step(2 * g + 2, 0)
        step(2 * g + 3, 1)

    wait_fetch(0)
    wait_fetch(1)

    tbase = pl.multiple_of(node_base + NFULL * CHUNK, 8)
    pltpu.sync_copy(ni_hbm.at[pl.ds(tbase, TAIL)], rows_v0.at[pl.ds(0, TAIL)])
    pltpu.sync_copy(idx_hbm.at[pl.ds(tbase, TAIL)], idxt_v)
    pltpu.sync_copy(rows_v0.at[pl.ds(0, TAIL)], sum_acc.at[idxt_v], add=True)
    pltpu.sync_copy(ones_v.at[pl.ds(0, TAIL)], cnt_acc.at[idxt_v], add=True)

    plsc.subcore_barrier()

    pltpu.sync_copy(sum_acc.at[pl.ds(r0, OUT_ROWS)],
                    sum_out.at[c, pl.ds(r0, OUT_ROWS)])
    pltpu.sync_copy(cnt_acc.at[pl.ds(r0, OUT_ROWS)], zc_v)
    for k in range(OUT_ROWS // 16):
        rows_v0[k // 8, pl.ds((k % 8) * 16, 16)] = zc_v[pl.ds(16 * k, 16)]
    cr0 = pl.multiple_of(s * CROWS, 8)
    pltpu.sync_copy(rows_v0.at[pl.ds(0, CROWS)],
                    cnt_out.at[c, pl.ds(cr0, CROWS)])


def _segment_sums(node_info, idx, zsum):
    mesh = plsc.VectorSubcoreMesh(core_axis_name="c", subcore_axis_name="s")
    f = functools.partial(
        pl.kernel,
        out_type=(jax.ShapeDtypeStruct((NC, MP, D), jnp.float32),
                  jax.ShapeDtypeStruct((NC, NS * CROWS, D), jnp.float32)),
        mesh=mesh,
        scratch_types=[
            pltpu.VMEM((CHUNK, D), jnp.float32),
            pltpu.VMEM((CHUNK, D), jnp.float32),
            pltpu.VMEM((CHUNK,), jnp.int32),
            pltpu.VMEM((CHUNK,), jnp.int32),
            pltpu.VMEM((TAIL,), jnp.int32),
            pltpu.VMEM((CHUNK,), jnp.float32),
            pltpu.VMEM((OUT_ROWS,), jnp.float32),
            pltpu.VMEM_SHARED((MP, D), jnp.float32),
            pltpu.VMEM_SHARED((MP,), jnp.float32),
        ] + [pltpu.SemaphoreType.DMA for _ in range(8)],
    )(_scatter_body)
    return f(node_info, idx, zsum)


def _update_body(sums_ref, cnt_ref, he_ref, wa_ref, wb_ref, b_ref, o_ref):
    ssum = sums_ref[0] + sums_ref[1]
    cnt = jnp.sum(cnt_ref[...], axis=1, keepdims=True)
    mean = ssum / jnp.maximum(cnt, 1.0)
    h = jnp.dot(mean, wa_ref[...], preferred_element_type=jnp.float32)
    h = h + jnp.dot(he_ref[...], wb_ref[...], preferred_element_type=jnp.float32)
    h = jnp.maximum(h + b_ref[...], 0.0)
    nrm = jnp.sqrt(jnp.sum(h * h, axis=-1, keepdims=True))
    o_ref[...] = h / jnp.maximum(nrm, 1e-12)


def _update(sums, cntsT, hyperedge, W_upd, b_upd):
    BR = 2000
    grid = (M // BR,)
    return pl.pallas_call(
        _update_body,
        grid=grid,
        in_specs=[
            pl.BlockSpec((NC, BR, D), lambda i: (0, i, 0)),
            pl.BlockSpec((BR, NC), lambda i: (i, 0)),
            pl.BlockSpec((BR, D), lambda i: (i, 0)),
            pl.BlockSpec((D, D), lambda i: (0, 0)),
            pl.BlockSpec((D, D), lambda i: (0, 0)),
            pl.BlockSpec((1, D), lambda i: (0, 0)),
        ],
        out_specs=pl.BlockSpec((BR, D), lambda i: (i, 0)),
        out_shape=jax.ShapeDtypeStruct((M, D), jnp.float32),
    )(sums, cntsT, hyperedge, W_upd[:D], W_upd[D:], b_upd.reshape(1, D))


def kernel(hyperedge, hyper_node, ve_affiliation, W_v2e, b_v2e, W_upd, b_upd):
    assert hyper_node.shape == (N, D) and hyperedge.shape == (M, D)
    node_info = _node_transform(hyper_node, W_v2e, b_v2e)
    idx = ve_affiliation[0]
    zsum = jnp.zeros((MP, D), jnp.float32)
    sums, cnts_wide = _segment_sums(node_info, idx, zsum)
    cntsT = (cnts_wide.reshape(NC, NS, CROWS * D)[:, :, :OUT_ROWS]
             .reshape(NC, MP).T)
    return _update(sums, cntsT, hyperedge, W_upd, b_upd)


# speedup vs baseline: 5.6963x; 1.0998x over previous
import functools

import jax
import jax.numpy as jnp
from jax import lax
from jax.experimental import pallas as pl
from jax.experimental.pallas import tpu as pltpu
from jax.experimental.pallas import tpu_sc as plsc

N = 320000
M = 10000
D = 128
MP = 10240

NC, NS = 2, 16
PER_TILE = N // (NC * NS)
CHUNK = 128
NFULL = PER_TILE // CHUNK
TAIL = PER_TILE - NFULL * CHUNK
OUT_ROWS = MP // NS
CROWS = 8


def _mm_relu_body(x_ref, w_ref, b_ref, o_ref):
    x = x_ref[...].astype(jnp.bfloat16)
    w = w_ref[...].astype(jnp.bfloat16)
    o_ref[...] = jnp.maximum(
        jnp.dot(x, w, preferred_element_type=jnp.float32) + b_ref[...], 0.0)


def _node_transform(hyper_node, W_v2e, b_v2e):
    BN = 6400
    grid = (N // BN,)
    return pl.pallas_call(
        _mm_relu_body,
        grid=grid,
        in_specs=[
            pl.BlockSpec((BN, D), lambda i: (i, 0)),
            pl.BlockSpec((D, D), lambda i: (0, 0)),
            pl.BlockSpec((1, D), lambda i: (0, 0)),
        ],
        out_specs=pl.BlockSpec((BN, D), lambda i: (i, 0)),
        out_shape=jax.ShapeDtypeStruct((N, D), jnp.float32),
    )(hyper_node, W_v2e, b_v2e.reshape(1, D))


def _scatter_body(ni_hbm, idx_hbm, zsum_hbm,
                  sum_out, cnt_out,
                  rows_v0, rows_v1, idx_v0, idx_v1, idxt_v,
                  ones_v, zc_v, sum_acc, cnt_acc,
                  rsem0, rsem1, isem0, isem1, ssem0, ssem1, csem0, csem1):
    c = lax.axis_index("c")
    s = lax.axis_index("s")
    rows = (rows_v0, rows_v1)
    idxs = (idx_v0, idx_v1)
    rsem = (rsem0, rsem1)
    isem = (isem0, isem1)
    ssem = (ssem0, ssem1)
    csem = (csem0, csem1)

    one16 = jnp.ones((16,), jnp.float32)
    zero16 = jnp.zeros((16,), jnp.float32)
    for k in range(CHUNK // 16):
        ones_v[pl.ds(16 * k, 16)] = one16
    for k in range(OUT_ROWS // 16):
        zc_v[pl.ds(16 * k, 16)] = zero16

    r0 = pl.multiple_of(s * OUT_ROWS, 8)
    pltpu.sync_copy(zsum_hbm.at[pl.ds(r0, OUT_ROWS)],
                    sum_acc.at[pl.ds(r0, OUT_ROWS)])
    pltpu.sync_copy(zc_v, cnt_acc.at[pl.ds(r0, OUT_ROWS)])
    plsc.subcore_barrier()

    node_base = c * (N // NC) + s * PER_TILE

    def start_fetch(k, b):
        kk = jnp.where(k < NFULL, k, 0)
        base = pl.multiple_of(node_base + kk * CHUNK, 8)
        pltpu.make_async_copy(ni_hbm.at[pl.ds(base, CHUNK)],
                              rows[b], rsem[b]).start()
        pltpu.make_async_copy(idx_hbm.at[pl.ds(base, CHUNK)],
                              idxs[b], isem[b]).start()

    def wait_fetch(b):
        pltpu.make_async_copy(ni_hbm.at[pl.ds(0, CHUNK)],
                              rows[b], rsem[b]).wait()
        pltpu.make_async_copy(idx_hbm.at[pl.ds(0, CHUNK)],
                              idxs[b], isem[b]).wait()

    def step(k, b):
        wait_fetch(b)
        sum_d = pltpu.make_async_copy(rows[b], sum_acc.at[idxs[b]], ssem[b])
        cnt_d = pltpu.make_async_copy(ones_v, cnt_acc.at[idxs[b]], csem[b])
        sum_d.start(add=True)
        cnt_d.start(add=True)
        sum_d.wait()
        cnt_d.wait()
        start_fetch(k + 2, b)

    start_fetch(0, 0)
    start_fetch(1, 1)
    step(0, 0)
    step(1, 1)

    @pl.loop(0, (NFULL - 2) // 2)
    def body(g):
        step(2 * g + 2, 0)
        step(2 * g + 3, 1)

    wait_fetch(0)
    wait_fetch(1)

    tbase = pl.multiple_of(node_base + NFULL * CHUNK, 8)
    pltpu.sync_copy(ni_hbm.at[pl.ds(tbase, TAIL)], rows_v0.at[pl.ds(0, TAIL)])
    pltpu.sync_copy(idx_hbm.at[pl.ds(tbase, TAIL)], idxt_v)
    pltpu.sync_copy(rows_v0.at[pl.ds(0, TAIL)], sum_acc.at[idxt_v], add=True)
    pltpu.sync_copy(ones_v.at[pl.ds(0, TAIL)], cnt_acc.at[idxt_v], add=True)

    plsc.subcore_barrier()

    pltpu.sync_copy(sum_acc.at[pl.ds(r0, OUT_ROWS)],
                    sum_out.at[c, pl.ds(r0, OUT_ROWS)])
    pltpu.sync_copy(cnt_acc.at[pl.ds(r0, OUT_ROWS)], zc_v)
    for k in range(OUT_ROWS // 16):
        rows_v0[k // 8, pl.ds((k % 8) * 16, 16)] = zc_v[pl.ds(16 * k, 16)]
    cr0 = pl.multiple_of(s * CROWS, 8)
    pltpu.sync_copy(rows_v0.at[pl.ds(0, CROWS)],
                    cnt_out.at[c, pl.ds(cr0, CROWS)])


def _segment_sums(node_info, idx, zsum):
    mesh = plsc.VectorSubcoreMesh(core_axis_name="c", subcore_axis_name="s")
    f = functools.partial(
        pl.kernel,
        out_type=(jax.ShapeDtypeStruct((NC, MP, D), jnp.float32),
                  jax.ShapeDtypeStruct((NC, NS * CROWS, D), jnp.float32)),
        mesh=mesh,
        scratch_types=[
            pltpu.VMEM((CHUNK, D), jnp.float32),
            pltpu.VMEM((CHUNK, D), jnp.float32),
            pltpu.VMEM((CHUNK,), jnp.int32),
            pltpu.VMEM((CHUNK,), jnp.int32),
            pltpu.VMEM((TAIL,), jnp.int32),
            pltpu.VMEM((CHUNK,), jnp.float32),
            pltpu.VMEM((OUT_ROWS,), jnp.float32),
            pltpu.VMEM_SHARED((MP, D), jnp.float32),
            pltpu.VMEM_SHARED((MP,), jnp.float32),
        ] + [pltpu.SemaphoreType.DMA for _ in range(8)],
    )(_scatter_body)
    return f(node_info, idx, zsum)


def _update_body(sums_ref, cnt_ref, he_ref, wa_ref, wb_ref, b_ref, o_ref):
    ssum = sums_ref[0] + sums_ref[1]
    cnt = jnp.sum(cnt_ref[...], axis=1, keepdims=True)
    mean = ssum / jnp.maximum(cnt, 1.0)
    h = jnp.dot(mean, wa_ref[...], preferred_element_type=jnp.float32)
    h = h + jnp.dot(he_ref[...], wb_ref[...], preferred_element_type=jnp.float32)
    h = jnp.maximum(h + b_ref[...], 0.0)
    nrm = jnp.sqrt(jnp.sum(h * h, axis=-1, keepdims=True))
    o_ref[...] = h / jnp.maximum(nrm, 1e-12)


def _update(sums, cntsT, hyperedge, W_upd, b_upd):
    BR = 2000
    grid = (M // BR,)
    return pl.pallas_call(
        _update_body,
        grid=grid,
        in_specs=[
            pl.BlockSpec((NC, BR, D), lambda i: (0, i, 0)),
            pl.BlockSpec((BR, NC), lambda i: (i, 0)),
            pl.BlockSpec((BR, D), lambda i: (i, 0)),
            pl.BlockSpec((D, D), lambda i: (0, 0)),
            pl.BlockSpec((D, D), lambda i: (0, 0)),
            pl.BlockSpec((1, D), lambda i: (0, 0)),
        ],
        out_specs=pl.BlockSpec((BR, D), lambda i: (i, 0)),
        out_shape=jax.ShapeDtypeStruct((M, D), jnp.float32),
    )(sums, cntsT, hyperedge, W_upd[:D], W_upd[D:], b_upd.reshape(1, D))


def kernel(hyperedge, hyper_node, ve_affiliation, W_v2e, b_v2e, W_upd, b_upd):
    assert hyper_node.shape == (N, D) and hyperedge.shape == (M, D)
    node_info = _node_transform(hyper_node, W_v2e, b_v2e)
    idx = ve_affiliation[0]
    zsum = jnp.zeros((MP, D), jnp.float32)
    sums, cnts_wide = _segment_sums(node_info, idx, zsum)
    cntsT = (cnts_wide.reshape(NC, NS, CROWS * D)[:, :, :OUT_ROWS]
             .reshape(NC, MP).T)
    return _update(sums, cntsT, hyperedge, W_upd, b_upd)


# speedup vs baseline: 5.7614x; 1.0114x over previous
import functools

import jax
import jax.numpy as jnp
from jax import lax
from jax.experimental import pallas as pl
from jax.experimental.pallas import tpu as pltpu
from jax.experimental.pallas import tpu_sc as plsc

N = 320000
M = 10000
D = 128
MP = 10240

NC, NS = 2, 16
NSPLIT = 2
NH = N // NSPLIT
CHUNK = 128
OUT_ROWS = MP // NS
CROWS = 8

PER_TILE = NH // (NC * NS)
NFULL = PER_TILE // CHUNK
TAIL = PER_TILE - NFULL * CHUNK
NPRE = 2 + (NFULL - 2) % 2
NPAIR = (NFULL - NPRE) // 2


def _mm_relu_body(x_ref, w_ref, b_ref, o_ref):
    x = x_ref[...].astype(jnp.bfloat16)
    w = w_ref[...].astype(jnp.bfloat16)
    o_ref[...] = jnp.maximum(
        jnp.dot(x, w, preferred_element_type=jnp.float32) + b_ref[...], 0.0)


def _node_transform(hyper_node, W_v2e, b_v2e, half):
    BN = 6400
    grid = (NH // BN,)
    off = half * (NH // BN)
    return pl.pallas_call(
        _mm_relu_body,
        grid=grid,
        in_specs=[
            pl.BlockSpec((BN, D), lambda i: (i + off, 0)),
            pl.BlockSpec((D, D), lambda i: (0, 0)),
            pl.BlockSpec((1, D), lambda i: (0, 0)),
        ],
        out_specs=pl.BlockSpec((BN, D), lambda i: (i, 0)),
        out_shape=jax.ShapeDtypeStruct((NH, D), jnp.float32),
    )(hyper_node, W_v2e, b_v2e.reshape(1, D))


def _scatter_body(ni_hbm, idx_hbm, zsum_hbm,
                  sum_out, cnt_out,
                  rows_v0, rows_v1, idx_v0, idx_v1, idxt_v,
                  ones_v, zc_v, sum_acc, cnt_acc,
                  rsem0, rsem1, isem0, isem1, ssem0, ssem1, csem0, csem1):
    c = lax.axis_index("c")
    s = lax.axis_index("s")
    rows = (rows_v0, rows_v1)
    idxs = (idx_v0, idx_v1)
    rsem = (rsem0, rsem1)
    isem = (isem0, isem1)
    ssem = (ssem0, ssem1)
    csem = (csem0, csem1)

    one16 = jnp.ones((16,), jnp.float32)
    zero16 = jnp.zeros((16,), jnp.float32)
    for k in range(CHUNK // 16):
        ones_v[pl.ds(16 * k, 16)] = one16
    for k in range(OUT_ROWS // 16):
        zc_v[pl.ds(16 * k, 16)] = zero16

    r0 = pl.multiple_of(s * OUT_ROWS, 8)
    pltpu.sync_copy(zsum_hbm.at[pl.ds(r0, OUT_ROWS)],
                    sum_acc.at[pl.ds(r0, OUT_ROWS)])
    pltpu.sync_copy(zc_v, cnt_acc.at[pl.ds(r0, OUT_ROWS)])
    plsc.subcore_barrier()

    node_base = c * (NH // NC) + s * PER_TILE

    def start_fetch(k, b):
        kk = jnp.where(k < NFULL, k, 0)
        base = pl.multiple_of(node_base + kk * CHUNK, 8)
        pltpu.make_async_copy(ni_hbm.at[pl.ds(base, CHUNK)],
                              rows[b], rsem[b]).start()
        pltpu.make_async_copy(idx_hbm.at[pl.ds(base, CHUNK)],
                              idxs[b], isem[b]).start()

    def wait_fetch(b):
        pltpu.make_async_copy(ni_hbm.at[pl.ds(0, CHUNK)],
                              rows[b], rsem[b]).wait()
        pltpu.make_async_copy(idx_hbm.at[pl.ds(0, CHUNK)],
                              idxs[b], isem[b]).wait()

    def step(k, b):
        wait_fetch(b)
        sum_d = pltpu.make_async_copy(rows[b], sum_acc.at[idxs[b]], ssem[b])
        cnt_d = pltpu.make_async_copy(ones_v, cnt_acc.at[idxs[b]], csem[b])
        sum_d.start(add=True)
        cnt_d.start(add=True)
        sum_d.wait()
        cnt_d.wait()
        start_fetch(k + 2, b)

    start_fetch(0, 0)
    start_fetch(1, 1)
    for k in range(NPRE):
        step(k, k % 2)

    @pl.loop(0, NPAIR)
    def body(g):
        k1 = 2 * g + NPRE
        step(k1, NPRE % 2)
        step(k1 + 1, 1 - NPRE % 2)

    wait_fetch(0)
    wait_fetch(1)

    tbase = pl.multiple_of(node_base + NFULL * CHUNK, 8)
    pltpu.sync_copy(ni_hbm.at[pl.ds(tbase, TAIL)], rows_v0.at[pl.ds(0, TAIL)])
    pltpu.sync_copy(idx_hbm.at[pl.ds(tbase, TAIL)], idxt_v)
    pltpu.sync_copy(rows_v0.at[pl.ds(0, TAIL)], sum_acc.at[idxt_v], add=True)
    pltpu.sync_copy(ones_v.at[pl.ds(0, TAIL)], cnt_acc.at[idxt_v], add=True)

    plsc.subcore_barrier()

    pltpu.sync_copy(sum_acc.at[pl.ds(r0, OUT_ROWS)],
                    sum_out.at[c, pl.ds(r0, OUT_ROWS)])
    pltpu.sync_copy(cnt_acc.at[pl.ds(r0, OUT_ROWS)], zc_v)
    for k in range(OUT_ROWS // 16):
        rows_v0[k // 8, pl.ds((k % 8) * 16, 16)] = zc_v[pl.ds(16 * k, 16)]
    cr0 = pl.multiple_of(s * CROWS, 8)
    pltpu.sync_copy(rows_v0.at[pl.ds(0, CROWS)],
                    cnt_out.at[c, pl.ds(cr0, CROWS)])


def _segment_sums(node_info, idx_half, zsum):
    mesh = plsc.VectorSubcoreMesh(core_axis_name="c", subcore_axis_name="s")
    f = functools.partial(
        pl.kernel,
        out_type=(jax.ShapeDtypeStruct((NC, MP, D), jnp.float32),
                  jax.ShapeDtypeStruct((NC, NS * CROWS, D), jnp.float32)),
        mesh=mesh,
        scratch_types=[
            pltpu.VMEM((CHUNK, D), jnp.float32),
            pltpu.VMEM((CHUNK, D), jnp.float32),
            pltpu.VMEM((CHUNK,), jnp.int32),
            pltpu.VMEM((CHUNK,), jnp.int32),
            pltpu.VMEM((TAIL,), jnp.int32),
            pltpu.VMEM((CHUNK,), jnp.float32),
            pltpu.VMEM((OUT_ROWS,), jnp.float32),
            pltpu.VMEM_SHARED((MP, D), jnp.float32),
            pltpu.VMEM_SHARED((MP,), jnp.float32),
        ] + [pltpu.SemaphoreType.DMA for _ in range(8)],
    )(_scatter_body)
    return f(node_info, idx_half, zsum)


def _update_body(sa_ref, sb_ref, cnt_ref, he_ref, wa_ref, wb_ref, b_ref,
                 o_ref):
    ssum = sa_ref[0] + sa_ref[1] + sb_ref[0] + sb_ref[1]
    cnt = jnp.sum(cnt_ref[...], axis=1, keepdims=True)
    mean = ssum / jnp.maximum(cnt, 1.0)
    h = jnp.dot(mean, wa_ref[...], preferred_element_type=jnp.float32)
    h = h + jnp.dot(he_ref[...], wb_ref[...], preferred_element_type=jnp.float32)
    h = jnp.maximum(h + b_ref[...], 0.0)
    nrm = jnp.sqrt(jnp.sum(h * h, axis=-1, keepdims=True))
    o_ref[...] = h / jnp.maximum(nrm, 1e-12)


def _update(sums_a, sums_b, cntsT, hyperedge, W_upd, b_upd):
    BR = 2000
    grid = (M // BR,)
    return pl.pallas_call(
        _update_body,
        grid=grid,
        in_specs=[
            pl.BlockSpec((NC, BR, D), lambda i: (0, i, 0)),
            pl.BlockSpec((NC, BR, D), lambda i: (0, i, 0)),
            pl.BlockSpec((BR, NSPLIT * NC), lambda i: (i, 0)),
            pl.BlockSpec((BR, D), lambda i: (i, 0)),
            pl.BlockSpec((D, D), lambda i: (0, 0)),
            pl.BlockSpec((D, D), lambda i: (0, 0)),
            pl.BlockSpec((1, D), lambda i: (0, 0)),
        ],
        out_specs=pl.BlockSpec((BR, D), lambda i: (i, 0)),
        out_shape=jax.ShapeDtypeStruct((M, D), jnp.float32),
    )(sums_a, sums_b, cntsT, hyperedge, W_upd[:D], W_upd[D:],
      b_upd.reshape(1, D))


def _counts_from_wide(cnts_wide):
    return cnts_wide.reshape(NC, NS, CROWS * D)[:, :, :OUT_ROWS].reshape(
        NC, MP)


def kernel(hyperedge, hyper_node, ve_affiliation, W_v2e, b_v2e, W_upd, b_upd):
    assert hyper_node.shape == (N, D) and hyperedge.shape == (M, D)
    idx = ve_affiliation[0]
    zsum = jnp.zeros((MP, D), jnp.float32)

    ni_a = _node_transform(hyper_node, W_v2e, b_v2e, 0)
    ni_b = _node_transform(hyper_node, W_v2e, b_v2e, 1)
    sums_a, cw_a = _segment_sums(ni_a, idx[:NH], zsum)
    sums_b, cw_b = _segment_sums(ni_b, idx[NH:], zsum)

    cntsT = jnp.concatenate(
        [_counts_from_wide(cw_a), _counts_from_wide(cw_b)], axis=0).T
    return _update(sums_a, sums_b, cntsT, hyperedge, W_upd, b_upd)
